# baseline (speedup 1.0000x reference)
import functools
import jax
import jax.numpy as jnp
from jax import lax
from jax.experimental import pallas as pl
from jax.experimental.pallas import tpu as pltpu, tpu_sc as plsc

H = 224
W = 224
HO = H // 2
WO = W // 2
NV = W // 16
NB = 256
NBV = NB // 16
IMGS = 768
IMGS_PER_WORKER = IMGS // 32
GROUP = 4

_MESH = plsc.VectorSubcoreMesh(core_axis_name="c", subcore_axis_name="s")


def _sortable_i32(xf):
    xb = plsc.bitcast(xf, jnp.int32)
    flip = (xb >> 31) | jnp.int32(-(2**31))
    return xb ^ flip


def _digit(key_i, shift):
    d = (plsc.bitcast(key_i, jnp.uint32) >> jnp.uint32(shift)) & jnp.uint32(NB - 1)
    return plsc.bitcast(d, jnp.int32)


def _body(x_hbm, out_hbm, ximg, oimg, *sets):
    wid = lax.axis_index("s") * 2 + lax.axis_index("c")
    iota = lax.iota(jnp.int32, 16)
    ones = jnp.ones((16,), jnp.int32)

    def sort_row(h, kA, kB, vA, vB, hist, base):
        for i in range(NBV):
            hist[pl.ds(16 * i, 16)] = jnp.zeros((16,), jnp.int32)
        for v in range(NV):
            key = _sortable_i32(ximg[h, pl.ds(16 * v, 16)])
            kA[pl.ds(16 * v, 16)] = key
            plsc.addupdate_scatter(hist, [_digit(key, 0)], ones)
        carry = jnp.int32(0)
        for i in range(NBV):
            hv = hist[pl.ds(16 * i, 16)]
            c = plsc.cumsum(hv)
            base[pl.ds(16 * i, 16)] = c - hv + carry
            carry = carry + c[15]
        for v in range(NV):
            k = kA[pl.ds(16 * v, 16)]
            d = _digit(k, 0)
            rc, last = plsc.scan_count(d)
            pos = plsc.load_gather(base, [d]) + rc - 1
            plsc.store_scatter(kB, [pos], k)
            plsc.store_scatter(vB, [pos], iota + jnp.int32(16 * v))
            plsc.addupdate_scatter(base, [d], rc, mask=last)
        for p, (sk, sv, dk, dv) in enumerate(
            [(kB, vB, kA, vA), (kA, vA, kB, vB), (kB, vB, kA, vA)], start=1
        ):
            shift = 8 * p
            for i in range(NBV):
                hist[pl.ds(16 * i, 16)] = jnp.zeros((16,), jnp.int32)
            for v in range(NV):
                plsc.addupdate_scatter(
                    hist, [_digit(sk[pl.ds(16 * v, 16)], shift)], ones)
            carry = jnp.int32(0)
            for i in range(NBV):
                hv = hist[pl.ds(16 * i, 16)]
                c = plsc.cumsum(hv)
                base[pl.ds(16 * i, 16)] = c - hv + carry
                carry = carry + c[15]
            for v in range(NV):
                k = sk[pl.ds(16 * v, 16)]
                val = sv[pl.ds(16 * v, 16)]
                d = _digit(k, shift)
                rc, last = plsc.scan_count(d)
                pos = plsc.load_gather(base, [d]) + rc - 1
                if p < 3:
                    plsc.store_scatter(dk, [pos], k)
                plsc.store_scatter(dv, [pos], val)
                plsc.addupdate_scatter(base, [d], rc, mask=last)
        s = []
        for m in range(WO // 16):
            idx = iota * 2 + jnp.int32(32 * m)
            e = plsc.load_gather(vA, [idx])
            o = plsc.load_gather(vA, [idx + 1])
            s.append(e + o)
        return s

    def img_body(jj, _):
        img = wid * IMGS_PER_WORKER + jj
        pltpu.sync_copy(x_hbm.at[pl.ds(img * H, H)], ximg)

        def group_body(g, _):
            srows = [
                sort_row(GROUP * g + r, *sets[6 * r : 6 * r + 6])
                for r in range(GROUP)
            ]
            for pr in range(GROUP // 2):
                hp = (GROUP // 2) * g + pr
                s0, s1 = srows[2 * pr], srows[2 * pr + 1]
                for m in range(WO // 16):
                    tot = (s0[m] + s1[m]).astype(jnp.float32) * 0.25
                    oimg[pl.ds(hp * WO + 16 * m, 16)] = tot
            return 0

        lax.fori_loop(0, H // GROUP, group_body, 0)
        pltpu.sync_copy(oimg, out_hbm.at[img])
        return 0

    lax.fori_loop(0, IMGS_PER_WORKER, img_body, 0)


_ROW_SET = [
    pltpu.VMEM((W,), jnp.int32),
    pltpu.VMEM((W,), jnp.int32),
    pltpu.VMEM((W,), jnp.int32),
    pltpu.VMEM((W,), jnp.int32),
    pltpu.VMEM((NB,), jnp.int32),
    pltpu.VMEM((NB,), jnp.int32),
]


@functools.partial(
    pl.kernel,
    out_type=jax.ShapeDtypeStruct((IMGS, HO * WO), jnp.float32),
    mesh=_MESH,
    compiler_params=pltpu.CompilerParams(needs_layout_passes=False),
    scratch_types=[
        pltpu.VMEM((H, W), jnp.float32),
        pltpu.VMEM((HO * WO,), jnp.float32),
    ] + GROUP * _ROW_SET,
)
def _sc_kernel(x_hbm, out_hbm, *scratch):
    _body(x_hbm, out_hbm, *scratch)


@jax.jit
def kernel(x):
    b, c, h, w = x.shape
    xf = x.reshape(b * c * h, w)
    out = _sc_kernel(xf)
    return out.reshape(b, c, HO, WO)
